# Initial kernel scaffold; baseline (speedup 1.0000x reference)
import jax
import jax.numpy as jnp
from jax.experimental import pallas as pl


def kernel(x, edge_index, W1, b1, att1, bias1, W2, b2, att2, bias2, W3, b3, att3, bias3):
    raise NotImplementedError("write your pallas kernel here")


# speedup vs baseline: 84.9397x; 84.9397x over previous
import functools

import jax
import jax.numpy as jnp
from jax import lax
from jax.experimental import pallas as pl
from jax.experimental.pallas import tpu as pltpu
from jax.experimental.pallas import tpu_sc as plsc

N = 10000
NPAD = 10112
E = 320000
WIN = 128
NTILES = 16
EP_ALL = 331776
P = 128
NEG = 0.2
F32 = jnp.float32


def _leaky(v):
    return jnp.where(v > 0, v, NEG * v)


def _make_sc(epc, tabrows, interpret=False):
    mesh = plsc.VectorSubcoreMesh(
        core_axis_name="c", subcore_axis_name="s", num_cores=2, num_subcores=16
    )
    rpt = NPAD // NTILES
    ept = epc // NTILES
    nwin = ept // WIN

    def body(tab_hbm, src_hbm, dst_hbm, out_hbm, acc_s, sbuf, dbuf, rbuf,
             zbuf, sem):
        cid = lax.axis_index("c")
        sid = lax.axis_index("s")
        for i in range(8):
            for j in range(8):
                zbuf[i, pl.ds(j * 16, 16)] = jnp.zeros((16,), F32)
        def zstep(r, carry):
            pltpu.sync_copy(zbuf, acc_s.at[pl.ds(sid * rpt + r * 8, 8)])
            return carry
        lax.fori_loop(0, rpt // 8, zstep, 0)
        plsc.subcore_barrier()

        def step(j, carry):
            eoff = cid * epc + sid * ept + j * WIN
            pltpu.sync_copy(src_hbm.at[pl.ds(eoff, WIN)], sbuf)
            pltpu.sync_copy(dst_hbm.at[pl.ds(eoff, WIN)], dbuf)
            pltpu.async_copy(tab_hbm.at[sbuf], rbuf, sem).wait()
            pltpu.sync_copy(rbuf, acc_s.at[dbuf], add=True)
            return carry

        lax.fori_loop(0, nwin, step, 0)
        plsc.subcore_barrier()
        pltpu.sync_copy(
            acc_s.at[pl.ds(sid * rpt, rpt)],
            out_hbm.at[pl.ds(cid * NPAD + sid * rpt, rpt)],
        )

    return pl.kernel(
        body,
        out_type=jax.ShapeDtypeStruct((2 * NPAD, P), F32),
        mesh=mesh,
        scratch_types=[
            pltpu.VMEM_SHARED((NPAD, P), F32),
            pltpu.VMEM((WIN,), jnp.int32),
            pltpu.VMEM((WIN,), jnp.int32),
            pltpu.VMEM((WIN, P), F32),
            pltpu.VMEM((8, P), F32),
            pltpu.SemaphoreType.DMA,
        ],
        interpret=interpret,
    )


def _dot(a, b):
    return jnp.dot(a, b, preferred_element_type=F32)


def _attn(h, amat):
    alpha = _dot(h, amat)
    al = _leaky(alpha)
    return jnp.exp(al - jnp.max(al, axis=0, keepdims=True))


def _store_table(out_ref, row0, half):
    out_ref[row0:row0 + N, :] = half
    out_ref[row0 + N:row0 + NPAD, :] = jnp.zeros((NPAD - N, P), F32)


def _tc1_body(x_ref, w_ref, b_ref, amat_ref, rmat_ref, out_ref):
    h = _dot(x_ref[...], w_ref[...]) + b_ref[...]
    expa = _attn(h, amat_ref[...])
    g = h * _dot(expa, rmat_ref[...])
    ones = jnp.ones((N, 1), F32)
    zpad = jnp.zeros((N, P - 69), F32)
    for c in (0, 1):
        half = jnp.concatenate(
            [g[:, 64 * c:64 * c + 64], expa[:, 4 * c:4 * c + 4], ones, zpad],
            axis=1)
        _store_table(out_ref, c * NPAD, half)


def _tc2_body(acc_ref, bias1_ref, w2_ref, b2_ref, amat_ref, rmat_ref,
              rden1_ref, out_ref):
    a0 = acc_ref[:N, :]
    a1 = acc_ref[NPAD:NPAD + N, :]
    num = jnp.concatenate([a0[:, :64], a1[:, :64]], axis=1)
    den = jnp.concatenate([a0[:, 64:68], a1[:, 64:68]], axis=1)
    cnt = jnp.maximum(a0[:, 68:69], 1.0)
    x2 = _leaky(num / _dot(den, rden1_ref[...]) / cnt + bias1_ref[...])
    h2 = _dot(x2, w2_ref[...]) + b2_ref[...]
    expa2 = _attn(h2, amat_ref[...])
    g2 = h2 * _dot(expa2, rmat_ref[...])
    ones = jnp.ones((N, 1), F32)
    half = jnp.concatenate(
        [g2, expa2, ones, jnp.zeros((N, P - 73), F32)], axis=1)
    _store_table(out_ref, 0, half)


def _tc3_body(acc_ref, bias2_ref, w3_ref, b3_ref, att3_ref, rden2_ref, out_ref):
    a = acc_ref[:N, :] + acc_ref[NPAD:NPAD + N, :]
    num = a[:, :64]
    den = a[:, 64:72]
    cnt = jnp.maximum(a[:, 72:73], 1.0)
    x3 = _leaky(num / _dot(den, rden2_ref[...]) / cnt + bias2_ref[...])
    h3 = _dot(x3, w3_ref[...]) + b3_ref[...]
    alpha3 = _dot(h3, att3_ref[...])
    al3 = _leaky(alpha3)
    expa3 = jnp.exp(al3 - jnp.max(al3, axis=0, keepdims=True))
    g3 = h3 * expa3
    ones = jnp.ones((N, 1), F32)
    half = jnp.concatenate(
        [g3, expa3, ones, jnp.zeros((N, P - 12), F32)], axis=1)
    _store_table(out_ref, 0, half)


def _tc4_body(acc_ref, bias3_ref, out_ref):
    a = acc_ref[:N, :] + acc_ref[NPAD:NPAD + N, :]
    num = a[:, :10]
    den = a[:, 10:11]
    cnt = jnp.maximum(a[:, 11:12], 1.0)
    logit = num / den / cnt + bias3_ref[...]
    m = jnp.max(logit, axis=1, keepdims=True)
    lse = jnp.log(jnp.sum(jnp.exp(logit - m), axis=1, keepdims=True))
    out_ref[...] = logit - m - lse


def _build(interpret=False):
    tc = functools.partial(pl.pallas_call, interpret=interpret)
    tc1 = tc(_tc1_body, out_shape=jax.ShapeDtypeStruct((2 * NPAD, P), F32))
    tc2 = tc(_tc2_body, out_shape=jax.ShapeDtypeStruct((NPAD, P), F32))
    tc3 = tc(_tc3_body, out_shape=jax.ShapeDtypeStruct((NPAD, P), F32))
    tc4 = tc(_tc4_body, out_shape=jax.ShapeDtypeStruct((N, 10), F32))
    return tc1, tc2, tc3, tc4


_TC1, _TC2, _TC3, _TC4 = _build()
_make_sc_cached = functools.lru_cache(maxsize=None)(_make_sc)


def kernel(x, edge_index, W1, b1, att1, bias1, W2, b2, att2, bias2,
           W3, b3, att3, bias3):
    src = edge_index[0].astype(jnp.int32)
    dst = edge_index[1].astype(jnp.int32)
    loops = jnp.arange(N, dtype=jnp.int32)
    padv = N + (jnp.arange(EP_ALL - E - N, dtype=jnp.int32) % (NPAD - N))
    srcp = jnp.concatenate([src, loops, padv])
    dstp = jnp.concatenate([dst, loops, padv])
    src1 = jnp.concatenate([srcp, srcp + NPAD])
    dst1 = jnp.concatenate([dstp, dstp])

    eye8 = jnp.eye(8, dtype=F32)
    a1m = (eye8[:, None, :] * att1[0][:, :, None]).reshape(128, 8)
    a2m = (eye8[:, None, :] * att2[0][:, :, None]).reshape(64, 8)
    r1 = jnp.repeat(eye8, 16, axis=1)
    r2 = jnp.repeat(eye8, 8, axis=1)
    att3v = att3[0, 0][:, None]

    tab1 = _TC1(x, W1, b1.reshape(1, -1), a1m, r1)
    acc1 = _make_sc_cached(EP_ALL, 2 * NPAD)(tab1, src1, dst1)
    tab2 = _TC2(acc1, bias1.reshape(1, -1), W2, b2.reshape(1, -1), a2m, r2, r1)
    acc2 = _make_sc_cached(EP_ALL // 2, NPAD)(tab2, srcp, dstp)
    tab3 = _TC3(acc2, bias2.reshape(1, -1), W3, b3.reshape(1, -1), att3v, r2)
    acc3 = _make_sc_cached(EP_ALL // 2, NPAD)(tab3, srcp, dstp)
    return _TC4(acc3, bias3.reshape(1, -1))


# speedup vs baseline: 117.0964x; 1.3786x over previous
import functools

import jax
import jax.numpy as jnp
from jax import lax
from jax.experimental import pallas as pl
from jax.experimental.pallas import tpu as pltpu
from jax.experimental.pallas import tpu_sc as plsc

N = 10000
NPAD = 10112
E = 320000
WIN = 128
NTILES = 16
EP1 = 335872
EP23 = 344064
P = 128
NEG = 0.2
F32 = jnp.float32


def _leaky(v):
    return jnp.where(v > 0, v, NEG * v)


KGRP = 2


def _make_sc(epc, tabrows, interpret=False):
    mesh = plsc.VectorSubcoreMesh(
        core_axis_name="c", subcore_axis_name="s", num_cores=2, num_subcores=16
    )
    rpt = NPAD // NTILES
    ept = epc // NTILES
    nwin = ept // WIN

    def body(tab_hbm, src_hbm, dst_hbm, out_hbm, acc_s,
             sbuf0, sbuf1, dbuf0, dbuf1, rbuf, zbuf, isem, gsem, ssem):
        sbufs = (sbuf0, sbuf1)
        dbufs = (dbuf0, dbuf1)
        cid = lax.axis_index("c")
        sid = lax.axis_index("s")
        for i in range(8):
            for j in range(8):
                zbuf[i, pl.ds(j * 16, 16)] = jnp.zeros((16,), F32)
        def zstep(r, carry):
            pltpu.sync_copy(zbuf, acc_s.at[pl.ds(sid * rpt + r * 8, 8)])
            return carry
        lax.fori_loop(0, rpt // 8, zstep, 0)
        ebase = cid * epc + sid * ept
        plsc.subcore_barrier()

        def group(g, carry):
            waits = []
            for b in range(KGRP):
                w = g * KGRP + b
                waits.append(pltpu.async_copy(
                    src_hbm.at[pl.ds(ebase + w * WIN, WIN)], sbufs[b], isem))
                waits.append(pltpu.async_copy(
                    dst_hbm.at[pl.ds(ebase + w * WIN, WIN)], dbufs[b], isem))
            for d in waits:
                d.wait()
            waits = []
            for b in range(KGRP):
                waits.append(pltpu.async_copy(
                    tab_hbm.at[sbufs[b]], rbuf.at[pl.ds(b * WIN, WIN)], gsem))
            for d in waits:
                d.wait()
            waits = []
            for b in range(KGRP):
                waits.append(pltpu.async_copy(
                    rbuf.at[pl.ds(b * WIN, WIN)], acc_s.at[dbufs[b]], ssem,
                    add=True))
            for d in waits:
                d.wait()
            return carry

        lax.fori_loop(0, nwin // KGRP, group, 0)
        plsc.subcore_barrier()
        pltpu.sync_copy(
            acc_s.at[pl.ds(sid * rpt, rpt)],
            out_hbm.at[pl.ds(cid * NPAD + sid * rpt, rpt)],
        )

    return pl.kernel(
        body,
        out_type=jax.ShapeDtypeStruct((2 * NPAD, P), F32),
        mesh=mesh,
        scratch_types=[
            pltpu.VMEM_SHARED((NPAD, P), F32),
            pltpu.VMEM((WIN,), jnp.int32),
            pltpu.VMEM((WIN,), jnp.int32),
            pltpu.VMEM((WIN,), jnp.int32),
            pltpu.VMEM((WIN,), jnp.int32),
            pltpu.VMEM((KGRP * WIN, P), F32),
            pltpu.VMEM((8, P), F32),
            pltpu.SemaphoreType.DMA,
            pltpu.SemaphoreType.DMA,
            pltpu.SemaphoreType.DMA,
        ],
        interpret=interpret,
    )


def _dot(a, b):
    return jnp.dot(a, b, preferred_element_type=F32)


def _attn(h, amat):
    alpha = _dot(h, amat)
    al = _leaky(alpha)
    return jnp.exp(al - jnp.max(al, axis=0, keepdims=True))


def _store_table(out_ref, row0, half):
    out_ref[row0:row0 + N, :] = half
    out_ref[row0 + N:row0 + NPAD, :] = jnp.zeros((NPAD - N, P), F32)


def _tc1_body(x_ref, w_ref, b_ref, amat_ref, rmat_ref, out_ref):
    h = _dot(x_ref[...], w_ref[...]) + b_ref[...]
    expa = _attn(h, amat_ref[...])
    g = h * _dot(expa, rmat_ref[...])
    ones = jnp.ones((N, 1), F32)
    zpad = jnp.zeros((N, P - 69), F32)
    for c in (0, 1):
        half = jnp.concatenate(
            [g[:, 64 * c:64 * c + 64], expa[:, 4 * c:4 * c + 4], ones, zpad],
            axis=1)
        _store_table(out_ref, c * NPAD, half)


def _tc2_body(acc_ref, bias1_ref, w2_ref, b2_ref, amat_ref, rmat_ref,
              rden1_ref, out_ref):
    a0 = acc_ref[:N, :]
    a1 = acc_ref[NPAD:NPAD + N, :]
    num = jnp.concatenate([a0[:, :64], a1[:, :64]], axis=1)
    den = jnp.concatenate([a0[:, 64:68], a1[:, 64:68]], axis=1)
    cnt = jnp.maximum(a0[:, 68:69], 1.0)
    x2 = _leaky(num / _dot(den, rden1_ref[...]) / cnt + bias1_ref[...])
    h2 = _dot(x2, w2_ref[...]) + b2_ref[...]
    expa2 = _attn(h2, amat_ref[...])
    g2 = h2 * _dot(expa2, rmat_ref[...])
    ones = jnp.ones((N, 1), F32)
    half = jnp.concatenate(
        [g2, expa2, ones, jnp.zeros((N, P - 73), F32)], axis=1)
    _store_table(out_ref, 0, half)


def _tc3_body(acc_ref, bias2_ref, w3_ref, b3_ref, att3_ref, rden2_ref, out_ref):
    a = acc_ref[:N, :] + acc_ref[NPAD:NPAD + N, :]
    num = a[:, :64]
    den = a[:, 64:72]
    cnt = jnp.maximum(a[:, 72:73], 1.0)
    x3 = _leaky(num / _dot(den, rden2_ref[...]) / cnt + bias2_ref[...])
    h3 = _dot(x3, w3_ref[...]) + b3_ref[...]
    alpha3 = _dot(h3, att3_ref[...])
    al3 = _leaky(alpha3)
    expa3 = jnp.exp(al3 - jnp.max(al3, axis=0, keepdims=True))
    g3 = h3 * expa3
    ones = jnp.ones((N, 1), F32)
    half = jnp.concatenate(
        [g3, expa3, ones, jnp.zeros((N, P - 12), F32)], axis=1)
    _store_table(out_ref, 0, half)


def _tc4_body(acc_ref, bias3_ref, out_ref):
    a = acc_ref[:N, :] + acc_ref[NPAD:NPAD + N, :]
    num = a[:, :10]
    den = a[:, 10:11]
    cnt = jnp.maximum(a[:, 11:12], 1.0)
    logit = num / den / cnt + bias3_ref[...]
    m = jnp.max(logit, axis=1, keepdims=True)
    lse = jnp.log(jnp.sum(jnp.exp(logit - m), axis=1, keepdims=True))
    out_ref[...] = logit - m - lse


def _build(interpret=False):
    tc = functools.partial(pl.pallas_call, interpret=interpret)
    tc1 = tc(_tc1_body, out_shape=jax.ShapeDtypeStruct((2 * NPAD, P), F32))
    tc2 = tc(_tc2_body, out_shape=jax.ShapeDtypeStruct((NPAD, P), F32))
    tc3 = tc(_tc3_body, out_shape=jax.ShapeDtypeStruct((NPAD, P), F32))
    tc4 = tc(_tc4_body, out_shape=jax.ShapeDtypeStruct((N, 10), F32))
    return tc1, tc2, tc3, tc4


_TC1, _TC2, _TC3, _TC4 = _build()
_make_sc_cached = functools.lru_cache(maxsize=None)(_make_sc)


def kernel(x, edge_index, W1, b1, att1, bias1, W2, b2, att2, bias2,
           W3, b3, att3, bias3):
    src = edge_index[0].astype(jnp.int32)
    dst = edge_index[1].astype(jnp.int32)
    loops = jnp.arange(N, dtype=jnp.int32)
    bsrc = jnp.concatenate([src, loops])
    bdst = jnp.concatenate([dst, loops])
    pad1 = N + (jnp.arange(EP1 - E - N, dtype=jnp.int32) % (NPAD - N))
    pad2 = N + (jnp.arange(EP23 - E - N, dtype=jnp.int32) % (NPAD - N))
    sl1 = jnp.concatenate([bsrc, pad1])
    dl1 = jnp.concatenate([bdst, pad1])
    src1 = jnp.concatenate([sl1, sl1 + NPAD])
    dst1 = jnp.concatenate([dl1, dl1])
    srcp = jnp.concatenate([bsrc, pad2])
    dstp = jnp.concatenate([bdst, pad2])

    eye8 = jnp.eye(8, dtype=F32)
    a1m = (eye8[:, None, :] * att1[0][:, :, None]).reshape(128, 8)
    a2m = (eye8[:, None, :] * att2[0][:, :, None]).reshape(64, 8)
    r1 = jnp.repeat(eye8, 16, axis=1)
    r2 = jnp.repeat(eye8, 8, axis=1)
    att3v = att3[0, 0][:, None]

    tab1 = _TC1(x, W1, b1.reshape(1, -1), a1m, r1)
    acc1 = _make_sc_cached(EP1, 2 * NPAD)(tab1, src1, dst1)
    tab2 = _TC2(acc1, bias1.reshape(1, -1), W2, b2.reshape(1, -1), a2m, r2, r1)
    acc2 = _make_sc_cached(EP23 // 2, NPAD)(tab2, srcp, dstp)
    tab3 = _TC3(acc2, bias2.reshape(1, -1), W3, b3.reshape(1, -1), att3v, r2)
    acc3 = _make_sc_cached(EP23 // 2, NPAD)(tab3, srcp, dstp)
    return _TC4(acc3, bias3.reshape(1, -1))


# speedup vs baseline: 143.4482x; 1.2250x over previous
import functools

import jax
import jax.numpy as jnp
from jax import lax
from jax.experimental import pallas as pl
from jax.experimental.pallas import tpu as pltpu
from jax.experimental.pallas import tpu_sc as plsc

N = 10000
NPAD = 10112
E = 320000
WIN = 128
NTILES = 16
EP1 = 335872
EP23 = 344064
P = 128
NEG = 0.2
F32 = jnp.float32


def _leaky(v):
    return jnp.where(v > 0, v, NEG * v)


def _make_sc(epc, tabrows, interpret=False):
    mesh = plsc.VectorSubcoreMesh(
        core_axis_name="c", subcore_axis_name="s", num_cores=2, num_subcores=16
    )
    rpt = NPAD // NTILES
    ept = epc // NTILES
    nwin = ept // WIN

    def body(tab_hbm, src_hbm, dst_hbm, out_hbm, acc_s,
             sa0, sa1, sb0, sb1, da0, da1, db0, db1, rbuf0, rbuf1,
             zbuf, isem, gsem, ssem):
        cid = lax.axis_index("c")
        sid = lax.axis_index("s")
        for i in range(8):
            for j in range(8):
                zbuf[i, pl.ds(j * 16, 16)] = jnp.zeros((16,), F32)
        def zstep(r, carry):
            pltpu.sync_copy(zbuf, acc_s.at[pl.ds(sid * rpt + r * 8, 8)])
            return carry
        lax.fori_loop(0, rpt // 8, zstep, 0)
        ebase = cid * epc + sid * ept
        plsc.subcore_barrier()

        for j in range(8):
            db0[pl.ds(j * 16, 16)] = jnp.full((16,), N, jnp.int32)
            db1[pl.ds(j * 16, 16)] = jnp.full((16,), N, jnp.int32)
        pltpu.async_copy(rbuf0, acc_s.at[db0], ssem, add=True)
        pltpu.async_copy(rbuf1, acc_s.at[db1], ssem, add=True)
        pltpu.async_copy(src_hbm.at[pl.ds(ebase, WIN)], sa0, isem)
        pltpu.async_copy(src_hbm.at[pl.ds(ebase + WIN, WIN)], sa1, isem)
        pltpu.async_copy(dst_hbm.at[pl.ds(ebase, WIN)], da0, isem)
        pltpu.async_copy(dst_hbm.at[pl.ds(ebase + WIN, WIN)], da1, isem)

        def drain_idx():
            for _ in range(4):
                pltpu.make_async_copy(
                    src_hbm.at[pl.ds(ebase, WIN)], sa0, isem).wait()

        def half(base, sx0, sx1, dx0, dx1, dpx0, dpx1, pre0, pre1):
            drain_idx()
            pltpu.make_async_copy(rbuf0, acc_s.at[dpx0], ssem).wait()
            pltpu.make_async_copy(rbuf1, acc_s.at[dpx1], ssem).wait()
            pltpu.async_copy(src_hbm.at[pl.ds(base + 2 * WIN, WIN)], pre0, isem)
            pltpu.async_copy(src_hbm.at[pl.ds(base + 3 * WIN, WIN)], pre1, isem)
            pltpu.async_copy(dst_hbm.at[pl.ds(base + 2 * WIN, WIN)], dpx0, isem)
            pltpu.async_copy(dst_hbm.at[pl.ds(base + 3 * WIN, WIN)], dpx1, isem)
            g0 = pltpu.async_copy(tab_hbm.at[sx0], rbuf0, gsem)
            g1 = pltpu.async_copy(tab_hbm.at[sx1], rbuf1, gsem)
            g0.wait()
            pltpu.async_copy(rbuf0, acc_s.at[dx0], ssem, add=True)
            g1.wait()
            pltpu.async_copy(rbuf1, acc_s.at[dx1], ssem, add=True)

        def it(g, carry):
            base = ebase + g * 4 * WIN
            half(base, sa0, sa1, da0, da1, db0, db1, sb0, sb1)
            half(base + 2 * WIN, sb0, sb1, db0, db1, da0, da1, sa0, sa1)
            return carry

        lax.fori_loop(0, nwin // 4, it, 0)
        drain_idx()
        pltpu.make_async_copy(rbuf0, acc_s.at[db0], ssem).wait()
        pltpu.make_async_copy(rbuf1, acc_s.at[db1], ssem).wait()
        plsc.subcore_barrier()
        pltpu.sync_copy(
            acc_s.at[pl.ds(sid * rpt, rpt)],
            out_hbm.at[pl.ds(cid * NPAD + sid * rpt, rpt)],
        )

    return pl.kernel(
        body,
        out_type=jax.ShapeDtypeStruct((2 * NPAD, P), F32),
        mesh=mesh,
        scratch_types=[
            pltpu.VMEM_SHARED((NPAD, P), F32),
            pltpu.VMEM((WIN,), jnp.int32),
            pltpu.VMEM((WIN,), jnp.int32),
            pltpu.VMEM((WIN,), jnp.int32),
            pltpu.VMEM((WIN,), jnp.int32),
            pltpu.VMEM((WIN,), jnp.int32),
            pltpu.VMEM((WIN,), jnp.int32),
            pltpu.VMEM((WIN,), jnp.int32),
            pltpu.VMEM((WIN,), jnp.int32),
            pltpu.VMEM((WIN, P), F32),
            pltpu.VMEM((WIN, P), F32),
            pltpu.VMEM((8, P), F32),
            pltpu.SemaphoreType.DMA,
            pltpu.SemaphoreType.DMA,
            pltpu.SemaphoreType.DMA,
        ],
        interpret=interpret,
    )


def _dot(a, b):
    return jnp.dot(a, b, preferred_element_type=F32)


def _attn(h, amat):
    alpha = _dot(h, amat)
    al = _leaky(alpha)
    return jnp.exp(al - jnp.max(al, axis=0, keepdims=True))


def _store_table(out_ref, row0, half):
    out_ref[row0:row0 + N, :] = half
    out_ref[row0 + N:row0 + NPAD, :] = jnp.zeros((NPAD - N, P), F32)


def _tc1_body(x_ref, w_ref, b_ref, amat_ref, rmat_ref, out_ref):
    h = _dot(x_ref[...], w_ref[...]) + b_ref[...]
    expa = _attn(h, amat_ref[...])
    g = h * _dot(expa, rmat_ref[...])
    ones = jnp.ones((N, 1), F32)
    zpad = jnp.zeros((N, P - 69), F32)
    for c in (0, 1):
        half = jnp.concatenate(
            [g[:, 64 * c:64 * c + 64], expa[:, 4 * c:4 * c + 4], ones, zpad],
            axis=1)
        _store_table(out_ref, c * NPAD, half)


def _tc2_body(acc_ref, bias1_ref, w2_ref, b2_ref, amat_ref, rmat_ref,
              rden1_ref, out_ref):
    a0 = acc_ref[:N, :]
    a1 = acc_ref[NPAD:NPAD + N, :]
    num = jnp.concatenate([a0[:, :64], a1[:, :64]], axis=1)
    den = jnp.concatenate([a0[:, 64:68], a1[:, 64:68]], axis=1)
    cnt = jnp.maximum(a0[:, 68:69], 1.0)
    x2 = _leaky(num / _dot(den, rden1_ref[...]) / cnt + bias1_ref[...])
    h2 = _dot(x2, w2_ref[...]) + b2_ref[...]
    expa2 = _attn(h2, amat_ref[...])
    g2 = h2 * _dot(expa2, rmat_ref[...])
    ones = jnp.ones((N, 1), F32)
    half = jnp.concatenate(
        [g2, expa2, ones, jnp.zeros((N, P - 73), F32)], axis=1)
    _store_table(out_ref, 0, half)


def _tc3_body(acc_ref, bias2_ref, w3_ref, b3_ref, att3_ref, rden2_ref, out_ref):
    a = acc_ref[:N, :] + acc_ref[NPAD:NPAD + N, :]
    num = a[:, :64]
    den = a[:, 64:72]
    cnt = jnp.maximum(a[:, 72:73], 1.0)
    x3 = _leaky(num / _dot(den, rden2_ref[...]) / cnt + bias2_ref[...])
    h3 = _dot(x3, w3_ref[...]) + b3_ref[...]
    alpha3 = _dot(h3, att3_ref[...])
    al3 = _leaky(alpha3)
    expa3 = jnp.exp(al3 - jnp.max(al3, axis=0, keepdims=True))
    g3 = h3 * expa3
    ones = jnp.ones((N, 1), F32)
    half = jnp.concatenate(
        [g3, expa3, ones, jnp.zeros((N, P - 12), F32)], axis=1)
    _store_table(out_ref, 0, half)


def _tc4_body(acc_ref, bias3_ref, out_ref):
    a = acc_ref[:N, :] + acc_ref[NPAD:NPAD + N, :]
    num = a[:, :10]
    den = a[:, 10:11]
    cnt = jnp.maximum(a[:, 11:12], 1.0)
    logit = num / den / cnt + bias3_ref[...]
    m = jnp.max(logit, axis=1, keepdims=True)
    lse = jnp.log(jnp.sum(jnp.exp(logit - m), axis=1, keepdims=True))
    out_ref[...] = logit - m - lse


def _build(interpret=False):
    tc = functools.partial(pl.pallas_call, interpret=interpret)
    tc1 = tc(_tc1_body, out_shape=jax.ShapeDtypeStruct((2 * NPAD, P), F32))
    tc2 = tc(_tc2_body, out_shape=jax.ShapeDtypeStruct((NPAD, P), F32))
    tc3 = tc(_tc3_body, out_shape=jax.ShapeDtypeStruct((NPAD, P), F32))
    tc4 = tc(_tc4_body, out_shape=jax.ShapeDtypeStruct((N, 10), F32))
    return tc1, tc2, tc3, tc4


_TC1, _TC2, _TC3, _TC4 = _build()
_make_sc_cached = functools.lru_cache(maxsize=None)(_make_sc)


def kernel(x, edge_index, W1, b1, att1, bias1, W2, b2, att2, bias2,
           W3, b3, att3, bias3):
    src = edge_index[0].astype(jnp.int32)
    dst = edge_index[1].astype(jnp.int32)
    loops = jnp.arange(N, dtype=jnp.int32)
    bsrc = jnp.concatenate([src, loops])
    bdst = jnp.concatenate([dst, loops])
    pad1 = N + (jnp.arange(EP1 - E - N, dtype=jnp.int32) % (NPAD - N))
    pad2 = N + (jnp.arange(EP23 - E - N, dtype=jnp.int32) % (NPAD - N))
    sl1 = jnp.concatenate([bsrc, pad1])
    dl1 = jnp.concatenate([bdst, pad1])
    tail = jnp.full((2 * WIN,), N, jnp.int32)
    src1 = jnp.concatenate([sl1, sl1 + NPAD, tail])
    dst1 = jnp.concatenate([dl1, dl1, tail])
    srcp = jnp.concatenate([bsrc, pad2, tail])
    dstp = jnp.concatenate([bdst, pad2, tail])

    eye8 = jnp.eye(8, dtype=F32)
    a1m = (eye8[:, None, :] * att1[0][:, :, None]).reshape(128, 8)
    a2m = (eye8[:, None, :] * att2[0][:, :, None]).reshape(64, 8)
    r1 = jnp.repeat(eye8, 16, axis=1)
    r2 = jnp.repeat(eye8, 8, axis=1)
    att3v = att3[0, 0][:, None]

    tab1 = _TC1(x, W1, b1.reshape(1, -1), a1m, r1)
    acc1 = _make_sc_cached(EP1, 2 * NPAD)(tab1, src1, dst1)
    tab2 = _TC2(acc1, bias1.reshape(1, -1), W2, b2.reshape(1, -1), a2m, r2, r1)
    acc2 = _make_sc_cached(EP23 // 2, NPAD)(tab2, srcp, dstp)
    tab3 = _TC3(acc2, bias2.reshape(1, -1), W3, b3.reshape(1, -1), att3v, r2)
    acc3 = _make_sc_cached(EP23 // 2, NPAD)(tab3, srcp, dstp)
    return _TC4(acc3, bias3.reshape(1, -1))


# speedup vs baseline: 146.1065x; 1.0185x over previous
import functools

import jax
import jax.numpy as jnp
from jax import lax
from jax.experimental import pallas as pl
from jax.experimental.pallas import tpu as pltpu
from jax.experimental.pallas import tpu_sc as plsc

N = 10000
NPAD = 10112
E = 320000
WIN = 128
NTILES = 16
EP1 = 335872
EP23 = 344064
P = 128
NEG = 0.2
F32 = jnp.float32


def _leaky(v):
    return jnp.where(v > 0, v, NEG * v)


def _make_sc(epc, tabrows, interpret=False):
    mesh = plsc.VectorSubcoreMesh(
        core_axis_name="c", subcore_axis_name="s", num_cores=2, num_subcores=16
    )
    rpt = NPAD // NTILES
    ept = epc // NTILES
    nwin = ept // WIN

    def body(tab_hbm, src_hbm, dst_hbm, out_hbm, acc_s,
             sa0, sa1, sb0, sb1, da0, da1, db0, db1, rbuf0, rbuf1,
             zbuf, isem, gsem, ssem, zsem):
        cid = lax.axis_index("c")
        sid = lax.axis_index("s")
        ebase = cid * epc + sid * ept
        pltpu.async_copy(src_hbm.at[pl.ds(ebase, WIN)], sa0, isem)
        pltpu.async_copy(src_hbm.at[pl.ds(ebase + WIN, WIN)], sa1, isem)
        pltpu.async_copy(dst_hbm.at[pl.ds(ebase, WIN)], da0, isem)
        pltpu.async_copy(dst_hbm.at[pl.ds(ebase + WIN, WIN)], da1, isem)
        for i in range(8):
            for j in range(8):
                zbuf[i, pl.ds(j * 16, 16)] = jnp.zeros((16,), F32)
        def zfire(r, carry):
            pltpu.async_copy(zbuf, acc_s.at[pl.ds(sid * rpt + r * 8, 8)], zsem)
            return carry
        lax.fori_loop(0, rpt // 8, zfire, 0)
        def zdrain(r, carry):
            pltpu.make_async_copy(
                zbuf, acc_s.at[pl.ds(sid * rpt, 8)], zsem).wait()
            return carry
        lax.fori_loop(0, rpt // 8, zdrain, 0)
        plsc.subcore_barrier()

        for j in range(8):
            db0[pl.ds(j * 16, 16)] = jnp.full((16,), N, jnp.int32)
            db1[pl.ds(j * 16, 16)] = jnp.full((16,), N, jnp.int32)
        pltpu.async_copy(rbuf0, acc_s.at[db0], ssem, add=True)
        pltpu.async_copy(rbuf1, acc_s.at[db1], ssem, add=True)

        def drain_idx():
            for _ in range(4):
                pltpu.make_async_copy(
                    src_hbm.at[pl.ds(ebase, WIN)], sa0, isem).wait()

        def half(base, sx0, sx1, dx0, dx1, dpx0, dpx1, pre0, pre1):
            drain_idx()
            pltpu.make_async_copy(rbuf0, acc_s.at[dpx0], ssem).wait()
            pltpu.make_async_copy(rbuf1, acc_s.at[dpx1], ssem).wait()
            pltpu.async_copy(src_hbm.at[pl.ds(base + 2 * WIN, WIN)], pre0, isem)
            pltpu.async_copy(src_hbm.at[pl.ds(base + 3 * WIN, WIN)], pre1, isem)
            pltpu.async_copy(dst_hbm.at[pl.ds(base + 2 * WIN, WIN)], dpx0, isem)
            pltpu.async_copy(dst_hbm.at[pl.ds(base + 3 * WIN, WIN)], dpx1, isem)
            g0 = pltpu.async_copy(tab_hbm.at[sx0], rbuf0, gsem)
            g1 = pltpu.async_copy(tab_hbm.at[sx1], rbuf1, gsem)
            g0.wait()
            pltpu.async_copy(rbuf0, acc_s.at[dx0], ssem, add=True)
            g1.wait()
            pltpu.async_copy(rbuf1, acc_s.at[dx1], ssem, add=True)

        def it(g, carry):
            base = ebase + g * 4 * WIN
            half(base, sa0, sa1, da0, da1, db0, db1, sb0, sb1)
            half(base + 2 * WIN, sb0, sb1, db0, db1, da0, da1, sa0, sa1)
            return carry

        lax.fori_loop(0, nwin // 4, it, 0)
        drain_idx()
        pltpu.make_async_copy(rbuf0, acc_s.at[db0], ssem).wait()
        pltpu.make_async_copy(rbuf1, acc_s.at[db1], ssem).wait()
        plsc.subcore_barrier()
        pltpu.sync_copy(
            acc_s.at[pl.ds(sid * rpt, rpt)],
            out_hbm.at[pl.ds(cid * NPAD + sid * rpt, rpt)],
        )

    return pl.kernel(
        body,
        out_type=jax.ShapeDtypeStruct((2 * NPAD, P), F32),
        mesh=mesh,
        scratch_types=[
            pltpu.VMEM_SHARED((NPAD, P), F32),
            pltpu.VMEM((WIN,), jnp.int32),
            pltpu.VMEM((WIN,), jnp.int32),
            pltpu.VMEM((WIN,), jnp.int32),
            pltpu.VMEM((WIN,), jnp.int32),
            pltpu.VMEM((WIN,), jnp.int32),
            pltpu.VMEM((WIN,), jnp.int32),
            pltpu.VMEM((WIN,), jnp.int32),
            pltpu.VMEM((WIN,), jnp.int32),
            pltpu.VMEM((WIN, P), F32),
            pltpu.VMEM((WIN, P), F32),
            pltpu.VMEM((8, P), F32),
            pltpu.SemaphoreType.DMA,
            pltpu.SemaphoreType.DMA,
            pltpu.SemaphoreType.DMA,
            pltpu.SemaphoreType.DMA,
        ],
        interpret=interpret,
    )


def _dot(a, b):
    return jnp.dot(a, b, preferred_element_type=F32)


def _attn(h, amat):
    alpha = _dot(h, amat)
    al = _leaky(alpha)
    return jnp.exp(al - jnp.max(al, axis=0, keepdims=True))


def _store_table(out_ref, row0, half):
    out_ref[row0:row0 + N, :] = half
    out_ref[row0 + N:row0 + NPAD, :] = jnp.zeros((NPAD - N, P), F32)


def _tc1_body(x_ref, w_ref, b_ref, amat_ref, rmat_ref, out_ref):
    h = _dot(x_ref[...], w_ref[...]) + b_ref[...]
    expa = _attn(h, amat_ref[...])
    g = h * _dot(expa, rmat_ref[...])
    ones = jnp.ones((N, 1), F32)
    zpad = jnp.zeros((N, P - 69), F32)
    for c in (0, 1):
        half = jnp.concatenate(
            [g[:, 64 * c:64 * c + 64], expa[:, 4 * c:4 * c + 4], ones, zpad],
            axis=1)
        _store_table(out_ref, c * NPAD, half)


def _tc2_body(acc_ref, bias1_ref, w2_ref, b2_ref, amat_ref, rmat_ref,
              rden1_ref, out_ref):
    a0 = acc_ref[:N, :]
    a1 = acc_ref[NPAD:NPAD + N, :]
    num = jnp.concatenate([a0[:, :64], a1[:, :64]], axis=1)
    den = jnp.concatenate([a0[:, 64:68], a1[:, 64:68]], axis=1)
    cnt = jnp.maximum(a0[:, 68:69], 1.0)
    x2 = _leaky(num / _dot(den, rden1_ref[...]) / cnt + bias1_ref[...])
    h2 = _dot(x2, w2_ref[...]) + b2_ref[...]
    expa2 = _attn(h2, amat_ref[...])
    g2 = h2 * _dot(expa2, rmat_ref[...])
    ones = jnp.ones((N, 1), F32)
    half = jnp.concatenate(
        [g2, expa2, ones, jnp.zeros((N, P - 73), F32)], axis=1)
    _store_table(out_ref, 0, half)


def _tc3_body(acc_ref, bias2_ref, w3_ref, b3_ref, att3_ref, rden2_ref, out_ref):
    a = acc_ref[:N, :] + acc_ref[NPAD:NPAD + N, :]
    num = a[:, :64]
    den = a[:, 64:72]
    cnt = jnp.maximum(a[:, 72:73], 1.0)
    x3 = _leaky(num / _dot(den, rden2_ref[...]) / cnt + bias2_ref[...])
    h3 = _dot(x3, w3_ref[...]) + b3_ref[...]
    alpha3 = _dot(h3, att3_ref[...])
    al3 = _leaky(alpha3)
    expa3 = jnp.exp(al3 - jnp.max(al3, axis=0, keepdims=True))
    g3 = h3 * expa3
    ones = jnp.ones((N, 1), F32)
    half = jnp.concatenate(
        [g3, expa3, ones, jnp.zeros((N, P - 12), F32)], axis=1)
    _store_table(out_ref, 0, half)


def _tc4_body(acc_ref, bias3_ref, out_ref):
    a = acc_ref[:N, :] + acc_ref[NPAD:NPAD + N, :]
    num = a[:, :10]
    den = a[:, 10:11]
    cnt = jnp.maximum(a[:, 11:12], 1.0)
    logit = num / den / cnt + bias3_ref[...]
    m = jnp.max(logit, axis=1, keepdims=True)
    lse = jnp.log(jnp.sum(jnp.exp(logit - m), axis=1, keepdims=True))
    out_ref[...] = logit - m - lse


def _build(interpret=False):
    tc = functools.partial(pl.pallas_call, interpret=interpret)
    tc1 = tc(_tc1_body, out_shape=jax.ShapeDtypeStruct((2 * NPAD, P), F32))
    tc2 = tc(_tc2_body, out_shape=jax.ShapeDtypeStruct((NPAD, P), F32))
    tc3 = tc(_tc3_body, out_shape=jax.ShapeDtypeStruct((NPAD, P), F32))
    tc4 = tc(_tc4_body, out_shape=jax.ShapeDtypeStruct((N, 10), F32))
    return tc1, tc2, tc3, tc4


_TC1, _TC2, _TC3, _TC4 = _build()
_make_sc_cached = functools.lru_cache(maxsize=None)(_make_sc)


def kernel(x, edge_index, W1, b1, att1, bias1, W2, b2, att2, bias2,
           W3, b3, att3, bias3):
    src = edge_index[0].astype(jnp.int32)
    dst = edge_index[1].astype(jnp.int32)
    loops = jnp.arange(N, dtype=jnp.int32)
    bsrc = jnp.concatenate([src, loops])
    bdst = jnp.concatenate([dst, loops])
    pad1 = N + (jnp.arange(EP1 - E - N, dtype=jnp.int32) % (NPAD - N))
    pad2 = N + (jnp.arange(EP23 - E - N, dtype=jnp.int32) % (NPAD - N))
    sl1 = jnp.concatenate([bsrc, pad1])
    dl1 = jnp.concatenate([bdst, pad1])
    tail = jnp.full((2 * WIN,), N, jnp.int32)
    src1 = jnp.concatenate([sl1, sl1 + NPAD, tail])
    dst1 = jnp.concatenate([dl1, dl1, tail])
    srcp = jnp.concatenate([bsrc, pad2, tail])
    dstp = jnp.concatenate([bdst, pad2, tail])

    eye8 = jnp.eye(8, dtype=F32)
    a1m = (eye8[:, None, :] * att1[0][:, :, None]).reshape(128, 8)
    a2m = (eye8[:, None, :] * att2[0][:, :, None]).reshape(64, 8)
    r1 = jnp.repeat(eye8, 16, axis=1)
    r2 = jnp.repeat(eye8, 8, axis=1)
    att3v = att3[0, 0][:, None]

    tab1 = _TC1(x, W1, b1.reshape(1, -1), a1m, r1)
    acc1 = _make_sc_cached(EP1, 2 * NPAD)(tab1, src1, dst1)
    tab2 = _TC2(acc1, bias1.reshape(1, -1), W2, b2.reshape(1, -1), a2m, r2, r1)
    acc2 = _make_sc_cached(EP23 // 2, NPAD)(tab2, srcp, dstp)
    tab3 = _TC3(acc2, bias2.reshape(1, -1), W3, b3.reshape(1, -1), att3v, r2)
    acc3 = _make_sc_cached(EP23 // 2, NPAD)(tab3, srcp, dstp)
    return _TC4(acc3, bias3.reshape(1, -1))
